# Initial kernel scaffold; baseline (speedup 1.0000x reference)
import jax
import jax.numpy as jnp
from jax.experimental import pallas as pl


def kernel(token_ids, embedding_table, W, b):
    raise NotImplementedError("write your pallas kernel here")


# speedup vs baseline: 2.2691x; 2.2691x over previous
import functools

import jax
import jax.numpy as jnp
from jax import lax
from jax.experimental import pallas as pl
from jax.experimental.pallas import tpu as pltpu
from jax.experimental.pallas import tpu_sc as plsc

_VOCAB = 1000000
_D = 32
_B = 4096
_L = 200
_NC = 2
_NS = 16
_NW = _NC * _NS
_SPW = _B // _NW
_CH = 8
_NCHUNK = _SPW // _CH
_TOK = _CH * _L
_UNROLL = 8


def _body(tok_hbm, table_hbm, w_hbm, b_hbm, out_hbm,
          idx_v, rows_v, out_v, w_v, b_v, sem):
    wid = lax.axis_index("s") * _NC + lax.axis_index("c")
    tok_base = wid * (_SPW * _L)

    pltpu.sync_copy(w_hbm, w_v)
    pltpu.sync_copy(b_hbm, b_v)
    w0 = w_v[pl.ds(0, 16)] * (1.0 / _L)
    w1 = w_v[pl.ds(16, 16)] * (1.0 / _L)
    bias = b_v[...][0]

    def chunk_body(c, carry):
        pltpu.sync_copy(tok_hbm.at[pl.ds(tok_base + c * _TOK, _TOK)], idx_v)
        pltpu.async_copy(table_hbm.at[idx_v], rows_v, sem).wait()

        def sent_body(j, carry2):
            def tok_body(tt, acc):
                a0, a1 = acc
                base = j * _L + tt * _UNROLL
                for u in range(_UNROLL):
                    a0 = a0 + rows_v[base + u, pl.ds(0, 16)]
                    a1 = a1 + rows_v[base + u, pl.ds(16, 16)]
                return a0, a1

            z = jnp.zeros((16,), jnp.float32)
            a0, a1 = lax.fori_loop(0, _L // _UNROLL, tok_body, (z, z))
            s = jnp.sum(a0 * w0 + a1 * w1) + bias
            out_v[c * _CH + j, pl.ds(0, 16)] = jnp.broadcast_to(s, (16,))
            return carry2

        lax.fori_loop(0, _CH, sent_body, 0)
        return carry

    lax.fori_loop(0, _NCHUNK, chunk_body, 0)
    pltpu.sync_copy(out_v, out_hbm.at[pl.ds(wid * _SPW, _SPW)])


def _postprocess(out_wide):
    return out_wide[:, :1]


def kernel(token_ids, embedding_table, W, b):
    tok_flat = token_ids.reshape(-1).astype(jnp.int32)
    w_flat = W.reshape(-1)
    b16 = jnp.broadcast_to(b, (16,)).astype(jnp.float32)
    mesh = plsc.VectorSubcoreMesh(core_axis_name="c", subcore_axis_name="s")
    run = functools.partial(
        pl.kernel,
        mesh=mesh,
        out_type=jax.ShapeDtypeStruct((_B, 16), jnp.float32),
        compiler_params=pltpu.CompilerParams(
            needs_layout_passes=False, use_tc_tiling_on_sc=False),
        scratch_types=[
            pltpu.VMEM((_TOK,), jnp.int32),
            pltpu.VMEM((_TOK, _D), jnp.float32),
            pltpu.VMEM((_SPW, 16), jnp.float32),
            pltpu.VMEM((_D,), jnp.float32),
            pltpu.VMEM((16,), jnp.float32),
            pltpu.SemaphoreType.DMA,
        ],
    )(_body)
    out = run(tok_flat, embedding_table, w_flat, b16)
    return _postprocess(out)


# speedup vs baseline: 16.2390x; 7.1565x over previous
import functools

import jax
import jax.numpy as jnp
from jax import lax
from jax.experimental import pallas as pl
from jax.experimental.pallas import tpu as pltpu
from jax.experimental.pallas import tpu_sc as plsc

_VOCAB = 1000000
_D = 32
_B = 4096
_L = 200
_NC = 2
_NS = 16
_NW = _NC * _NS
_SPW = _B // _NW
_BLKV = 65536
_GRID = (_VOCAB + _BLKV - 1) // _BLKV


def _mv_body(tT_ref, w_ref, p_ref):
    p_ref[...] = jnp.sum(tT_ref[...] * w_ref[...], axis=0)


def _matvec(tT, w2):
    return pl.pallas_call(
        _mv_body,
        grid=(_GRID,),
        in_specs=[
            pl.BlockSpec((_D, _BLKV), lambda i: (0, i)),
            pl.BlockSpec((_D, 1), lambda i: (0, 0)),
        ],
        out_specs=pl.BlockSpec((_BLKV,), lambda i: (i,)),
        out_shape=jax.ShapeDtypeStruct((_VOCAB,), jnp.float32),
        compiler_params=pltpu.CompilerParams(
            dimension_semantics=("arbitrary",),
        ),
    )(tT, w2)


_PSLICE = _VOCAB // 8


def _sc_body(tokT_hbm, p_hbm, b_hbm, out_hbm,
             p_sh, idx_v, vals_v, out_v, b_v, sem):
    cid = lax.axis_index("c")
    sid = lax.axis_index("s")
    wid = sid * _NC + cid

    @pl.when(sid < 8)
    def _stage_p():
        off = sid * _PSLICE
        pltpu.sync_copy(p_hbm.at[pl.ds(off, _PSLICE)],
                        p_sh.at[pl.ds(off, _PSLICE)])

    pltpu.sync_copy(b_hbm, b_v)
    col = wid * _SPW
    pltpu.sync_copy(tokT_hbm.at[:, pl.ds(col, _SPW)], idx_v)
    plsc.subcore_barrier()

    def fire(t, c):
        pltpu.async_copy(p_sh.at[idx_v.at[t]], vals_v.at[t], sem)
        return c

    lax.fori_loop(0, _L, fire, 0)

    def drain(t, c):
        pltpu.make_async_copy(p_sh.at[idx_v.at[0]], vals_v.at[0], sem).wait()
        return c

    lax.fori_loop(0, _L, drain, 0)

    bias = b_v[...][0]

    def tok_body(t, acc):
        return tuple(
            acc[g] + vals_v[t, pl.ds(16 * g, 16)] for g in range(_SPW // 16)
        )

    z = jnp.zeros((16,), jnp.float32)
    acc = lax.fori_loop(0, _L, tok_body, (z,) * (_SPW // 16))
    for g in range(_SPW // 16):
        out_v[pl.ds(16 * g, 16)] = acc[g] + bias
    pltpu.sync_copy(out_v, out_hbm.at[pl.ds(col, _SPW)])


def _sc_gather(tokT, p, b16):
    mesh = plsc.VectorSubcoreMesh(core_axis_name="c", subcore_axis_name="s")
    run = functools.partial(
        pl.kernel,
        mesh=mesh,
        out_type=jax.ShapeDtypeStruct((_B,), jnp.float32),
        scratch_types=[
            pltpu.VMEM_SHARED((_VOCAB,), jnp.float32),
            pltpu.VMEM((_L, _SPW), jnp.int32),
            pltpu.VMEM((_L, _SPW), jnp.float32),
            pltpu.VMEM((_SPW,), jnp.float32),
            pltpu.VMEM((16,), jnp.float32),
            pltpu.SemaphoreType.DMA,
        ],
        compiler_params=pltpu.CompilerParams(
            needs_layout_passes=False, use_tc_tiling_on_sc=False),
    )(_sc_body)
    return run(tokT, p, b16)


def kernel(token_ids, embedding_table, W, b):
    tT = embedding_table.T
    tokT = token_ids.T
    w2 = (W * (1.0 / _L)).astype(jnp.float32)
    b16 = jnp.broadcast_to(b, (16,)).astype(jnp.float32)
    p = _matvec(tT, w2)
    out = _sc_gather(tokT, p, b16)
    return out.reshape(_B, 1)


# speedup vs baseline: 17.3144x; 1.0662x over previous
import functools

import jax
import jax.numpy as jnp
from jax import lax
from jax.experimental import pallas as pl
from jax.experimental.pallas import tpu as pltpu
from jax.experimental.pallas import tpu_sc as plsc

_VOCAB = 1000000
_D = 32
_B = 4096
_L = 200
_NC = 2
_NS = 16
_NW = _NC * _NS
_SPW = _B // _NW
_BLKV = 65536
_VPAD = 1000448
_GRID = (_VPAD + _BLKV - 1) // _BLKV


def _mv_body(tT_ref, w_ref, p_ref):
    p_ref[...] = jnp.sum(tT_ref[...] * w_ref[...], axis=0)


def _matvec(tT, w2):
    return pl.pallas_call(
        _mv_body,
        grid=(_GRID,),
        in_specs=[
            pl.BlockSpec((_D, _BLKV), lambda i: (0, i)),
            pl.BlockSpec((_D, 1), lambda i: (0, 0)),
        ],
        out_specs=pl.BlockSpec((_BLKV,), lambda i: (i,)),
        out_shape=jax.ShapeDtypeStruct((_VPAD,), jnp.float32),
        compiler_params=pltpu.CompilerParams(
            dimension_semantics=("arbitrary",),
        ),
    )(tT, w2)


_PSLICE = _VPAD // 8


def _sc_body(tokT_hbm, p_hbm, b_hbm, out_hbm,
             p_sh, idx_v, vals_v, out_v, b_v, sem):
    cid = lax.axis_index("c")
    sid = lax.axis_index("s")
    wid = sid * _NC + cid

    @pl.when(sid < 8)
    def _stage_p():
        off = sid * _PSLICE
        pltpu.sync_copy(p_hbm.at[pl.ds(off, _PSLICE)],
                        p_sh.at[pl.ds(off, _PSLICE)])

    pltpu.sync_copy(b_hbm, b_v)
    col = wid * _SPW
    pltpu.sync_copy(tokT_hbm.at[:, pl.ds(col, _SPW)], idx_v)
    plsc.subcore_barrier()

    def fire(t, c):
        pltpu.async_copy(p_sh.at[idx_v.at[t]], vals_v.at[t], sem)
        return c

    lax.fori_loop(0, _L, fire, 0)

    def drain(t, c):
        pltpu.make_async_copy(p_sh.at[idx_v.at[0]], vals_v.at[0], sem).wait()
        return c

    lax.fori_loop(0, _L, drain, 0)

    bias = b_v[pl.ds(0, 16)][0]

    def tok_body(t, acc):
        return tuple(
            acc[g] + vals_v[t, pl.ds(16 * g, 16)] for g in range(_SPW // 16)
        )

    z = jnp.zeros((16,), jnp.float32)
    acc = lax.fori_loop(0, _L, tok_body, (z,) * (_SPW // 16))
    for g in range(_SPW // 16):
        out_v[pl.ds(16 * g, 16)] = acc[g] + bias
    pltpu.sync_copy(out_v, out_hbm.at[pl.ds(col, _SPW)])


def _sc_gather(tokT, p, b16):
    mesh = plsc.VectorSubcoreMesh(core_axis_name="c", subcore_axis_name="s")
    run = functools.partial(
        pl.kernel,
        mesh=mesh,
        out_type=jax.ShapeDtypeStruct((_B,), jnp.float32),
        scratch_types=[
            pltpu.VMEM_SHARED((_VPAD,), jnp.float32),
            pltpu.VMEM((_L, _SPW), jnp.int32),
            pltpu.VMEM((_L, _SPW), jnp.float32),
            pltpu.VMEM((_SPW,), jnp.float32),
            pltpu.VMEM((128,), jnp.float32),
            pltpu.SemaphoreType.DMA,
        ],
        compiler_params=pltpu.CompilerParams(
            needs_layout_passes=False, use_tc_tiling_on_sc=True),
    )(_sc_body)
    return run(tokT, p, b16)


def kernel(token_ids, embedding_table, W, b):
    tT = embedding_table.T
    tokT = token_ids.T
    w2 = (W * (1.0 / _L)).astype(jnp.float32)
    b16 = jnp.broadcast_to(b, (128,)).astype(jnp.float32)
    p = _matvec(tT, w2)
    out = _sc_gather(tokT, p, b16)
    return out.reshape(_B, 1)
